# baseline (speedup 1.0000x reference)
import functools

import jax
import jax.numpy as jnp
from jax import lax
from jax.experimental import pallas as pl
from jax.experimental.pallas import tpu as pltpu
from jax.experimental.pallas import tpu_sc as plsc

_DESIRED_SIZE = 448.0
_ALPHA = 0.5
_PENALTY_WEIGHT = 0.015
_EPS = 1e-7

_NC = 2
_NS = 16
_NW = _NC * _NS
_L = 16

_N_BOXES = 16 * 20000
_BPW = _N_BOXES // _NW
_GROUPS = _BPW // _L
_PRED_W = _BPW * 5
_TGT_W = _BPW * 4

_mesh = plsc.VectorSubcoreMesh(
    core_axis_name="c", subcore_axis_name="s", num_cores=_NC, num_subcores=_NS
)


@functools.partial(
    pl.kernel,
    out_type=jax.ShapeDtypeStruct((_NW, 4, _L), jnp.float32),
    mesh=_mesh,
    scratch_types=[
        pltpu.VMEM((_PRED_W,), jnp.float32),
        pltpu.VMEM((_TGT_W,), jnp.float32),
        pltpu.VMEM((4, _L), jnp.float32),
    ],
    compiler_params=pltpu.CompilerParams(needs_layout_passes=False),
)
def _loss_partials(pred_hbm, tgt_hbm, out_hbm, pred_v, tgt_v, acc_v):
    wid = lax.axis_index("s") * _NC + lax.axis_index("c")
    pltpu.sync_copy(pred_hbm.at[pl.ds(wid * _PRED_W, _PRED_W)], pred_v)
    pltpu.sync_copy(tgt_hbm.at[pl.ds(wid * _TGT_W, _TGT_W)], tgt_v)

    iota = lax.iota(jnp.int32, _L)
    p_base = iota * 5
    t_base = iota * 4
    zeros = jnp.zeros((_L,), jnp.float32)

    @plsc.parallel_loop(0, _GROUPS, 1, unroll=8, carry=(zeros, zeros, zeros))
    def _acc(g, carry):
        acc_d, acc_s, acc_p = carry
        pb = p_base + g * (5 * _L)
        tb = t_base + g * (4 * _L)
        x1 = plsc.load_gather(pred_v, [pb])
        y1 = plsc.load_gather(pred_v, [pb + 1])
        x2 = plsc.load_gather(pred_v, [pb + 2])
        y2 = plsc.load_gather(pred_v, [pb + 3])
        tx1 = plsc.load_gather(tgt_v, [tb])
        ty1 = plsc.load_gather(tgt_v, [tb + 1])
        tx2 = plsc.load_gather(tgt_v, [tb + 2])
        ty2 = plsc.load_gather(tgt_v, [tb + 3])

        pred_area = jnp.maximum(x2 - x1, 0.0) * jnp.maximum(y2 - y1, 0.0)
        tgt_area = jnp.maximum(tx2 - tx1, 0.0) * jnp.maximum(ty2 - ty1, 0.0)
        inter = jnp.maximum(jnp.minimum(x2, tx2) - jnp.maximum(x1, tx1), 0.0) * \
            jnp.maximum(jnp.minimum(y2, ty2) - jnp.maximum(y1, ty1), 0.0)
        union = pred_area + tgt_area - inter
        iou = inter / (union + _EPS)
        dx = (x1 + x2) * 0.5 - (tx1 + tx2) * 0.5
        dy = (y1 + y2) * 0.5 - (ty1 + ty2) * 0.5
        cdist = dx * dx + dy * dy
        ew = jnp.maximum(x2, tx2) - jnp.minimum(x1, tx1)
        eh = jnp.maximum(y2, ty2) - jnp.minimum(y1, ty1)
        diag = ew * ew + eh * eh
        dl = 1.0 - iou + cdist / (diag + _EPS)

        sl = zeros
        for p, t in ((x1, tx1), (y1, ty1), (x2, tx2), (y2, ty2)):
            d = p - t
            ad = jnp.abs(d)
            sl = sl + jnp.where(ad < 1.0, 0.5 * d * d, ad - 0.5)

        pen = jnp.maximum(x1 - x2, 0.0) + jnp.maximum(y1 - y2, 0.0)
        pen = pen + jnp.where(x1 >= x2 + 1.0, 1.0, 0.0)
        pen = pen + jnp.where(y1 >= y2 + 1.0, 1.0, 0.0)
        for v in (x1, y1, x2, y2):
            pen = pen + jnp.maximum(-v, 0.0) + jnp.maximum(v - _DESIRED_SIZE, 0.0)

        return acc_d + dl, acc_s + sl, acc_p + pen

    acc_d, acc_s, acc_p = _acc
    acc_v[0, :] = acc_d
    acc_v[1, :] = acc_s
    acc_v[2, :] = acc_p
    acc_v[3, :] = zeros
    pltpu.sync_copy(acc_v, out_hbm.at[wid])


def kernel(pred_boxes, target_boxes):
    pred_flat = pred_boxes.reshape(-1)
    tgt_flat = target_boxes.reshape(-1)
    parts = _loss_partials(pred_flat, tgt_flat)
    s = jnp.sum(parts, axis=(0, 2))
    dl = s[0] / _N_BOXES
    sl = s[1] / (_N_BOXES * 4)
    pen = s[2] / _DESIRED_SIZE
    return _ALPHA * dl + (1.0 - _ALPHA) * sl + _PENALTY_WEIGHT * pen


# speedup vs baseline: 6.2354x; 6.2354x over previous
import functools

import jax
import jax.numpy as jnp
from jax import lax
from jax.experimental import pallas as pl
from jax.experimental.pallas import tpu as pltpu
from jax.experimental.pallas import tpu_sc as plsc

_DESIRED_SIZE = 448.0
_ALPHA = 0.5
_PENALTY_WEIGHT = 0.015
_EPS = 1e-7

_NC = 2
_NS = 16
_NW = _NC * _NS
_L = 16

_N_BOXES = 16 * 20000
_BPW = _N_BOXES // _NW
_GROUPS = _BPW // _L

_mesh = plsc.VectorSubcoreMesh(
    core_axis_name="c", subcore_axis_name="s", num_cores=_NC, num_subcores=_NS
)


@functools.partial(
    pl.kernel,
    out_type=jax.ShapeDtypeStruct((_NW, 4, _L), jnp.float32),
    mesh=_mesh,
    scratch_types=[
        pltpu.VMEM((_BPW,), jnp.float32),
        pltpu.VMEM((_BPW,), jnp.float32),
        pltpu.VMEM((_BPW,), jnp.float32),
        pltpu.VMEM((_BPW,), jnp.float32),
        pltpu.VMEM((_BPW,), jnp.float32),
        pltpu.VMEM((_BPW,), jnp.float32),
        pltpu.VMEM((_BPW,), jnp.float32),
        pltpu.VMEM((_BPW,), jnp.float32),
        pltpu.VMEM((4, _L), jnp.float32),
    ],
)
def _loss_partials(pred_hbm, tgt_hbm, out_hbm,
                   x1v, y1v, x2v, y2v, tx1v, ty1v, tx2v, ty2v, acc_v):
    wid = lax.axis_index("s") * _NC + lax.axis_index("c")
    base = wid * _BPW
    for c, dst in enumerate((x1v, y1v, x2v, y2v)):
        pltpu.sync_copy(pred_hbm.at[pl.ds(c * _N_BOXES + base, _BPW)], dst)
    for c, dst in enumerate((tx1v, ty1v, tx2v, ty2v)):
        pltpu.sync_copy(tgt_hbm.at[pl.ds(c * _N_BOXES + base, _BPW)], dst)

    zeros = jnp.zeros((_L,), jnp.float32)

    @plsc.parallel_loop(0, _GROUPS, 1, unroll=8, carry=(zeros, zeros, zeros))
    def _acc(g, carry):
        acc_d, acc_s, acc_p = carry
        o = g * _L
        x1 = x1v[pl.ds(o, _L)]
        y1 = y1v[pl.ds(o, _L)]
        x2 = x2v[pl.ds(o, _L)]
        y2 = y2v[pl.ds(o, _L)]
        tx1 = tx1v[pl.ds(o, _L)]
        ty1 = ty1v[pl.ds(o, _L)]
        tx2 = tx2v[pl.ds(o, _L)]
        ty2 = ty2v[pl.ds(o, _L)]

        pred_area = jnp.maximum(x2 - x1, 0.0) * jnp.maximum(y2 - y1, 0.0)
        tgt_area = jnp.maximum(tx2 - tx1, 0.0) * jnp.maximum(ty2 - ty1, 0.0)
        inter = jnp.maximum(jnp.minimum(x2, tx2) - jnp.maximum(x1, tx1), 0.0) * \
            jnp.maximum(jnp.minimum(y2, ty2) - jnp.maximum(y1, ty1), 0.0)
        union_e = pred_area + tgt_area - inter + _EPS
        dx = (x1 + x2) * 0.5 - (tx1 + tx2) * 0.5
        dy = (y1 + y2) * 0.5 - (ty1 + ty2) * 0.5
        cdist = dx * dx + dy * dy
        ew = jnp.maximum(x2, tx2) - jnp.minimum(x1, tx1)
        eh = jnp.maximum(y2, ty2) - jnp.minimum(y1, ty1)
        diag_e = ew * ew + eh * eh + _EPS
        dl = 1.0 + (cdist * union_e - inter * diag_e) / (union_e * diag_e)

        sl = zeros
        for p, t in ((x1, tx1), (y1, ty1), (x2, tx2), (y2, ty2)):
            d = p - t
            ad = jnp.abs(d)
            sl = sl + jnp.where(ad < 1.0, 0.5 * d * d, ad - 0.5)

        pen = jnp.maximum(x1 - x2, 0.0) + jnp.maximum(y1 - y2, 0.0)
        pen = pen + jnp.where(x1 >= x2 + 1.0, 1.0, 0.0)
        pen = pen + jnp.where(y1 >= y2 + 1.0, 1.0, 0.0)
        for v in (x1, y1, x2, y2):
            pen = pen + jnp.maximum(-v, 0.0) + jnp.maximum(v - _DESIRED_SIZE, 0.0)

        return acc_d + dl, acc_s + sl, acc_p + pen

    acc_d, acc_s, acc_p = _acc
    acc_v[0, :] = acc_d
    acc_v[1, :] = acc_s
    acc_v[2, :] = acc_p
    acc_v[3, :] = zeros
    pltpu.sync_copy(acc_v, out_hbm.at[wid])


def kernel(pred_boxes, target_boxes):
    pred_cols = jnp.transpose(pred_boxes, (2, 0, 1)).reshape(5, -1)[:4].reshape(-1)
    tgt_cols = jnp.transpose(target_boxes, (2, 0, 1)).reshape(-1)
    parts = _loss_partials(pred_cols, tgt_cols)
    s = jnp.sum(parts, axis=(0, 2))
    dl = s[0] / _N_BOXES
    sl = s[1] / (_N_BOXES * 4)
    pen = s[2] / _DESIRED_SIZE
    return _ALPHA * dl + (1.0 - _ALPHA) * sl + _PENALTY_WEIGHT * pen
